# Initial kernel scaffold; baseline (speedup 1.0000x reference)
import jax
import jax.numpy as jnp
from jax.experimental import pallas as pl


def kernel(proj_features, prev_scenepoint_features, edge_src, edge_dst, ln1_w, ln1_b, W_l, b_l, W_r, b_r, att, out_bias, ln2_w, ln2_b, W_mlp, b_mlp):
    raise NotImplementedError("write your pallas kernel here")


# speedup vs baseline: 22.0887x; 22.0887x over previous
import functools

import jax
import jax.numpy as jnp
from jax import lax
from jax.experimental import pallas as pl
from jax.experimental.pallas import tpu as pltpu
from jax.experimental.pallas import tpu_sc as plsc

NC = 2
NS = 16
H = 8
HL = H // NC
C = 16
DH = HL * C
GATHER = 128
KSUB = 2
CE = GATHER * KSUB
PAD = 16


def _axis_index(name):
    return lax.axis_index(name)


def _mm_bias_body(x_ref, w_ref, b_ref, o0_ref, o1_ref):
    r = jnp.dot(x_ref[...], w_ref[...], preferred_element_type=jnp.float32)
    r = r + b_ref[...]
    o0_ref[...] = r[:, :DH]
    o1_ref[...] = r[:, DH:]


def _ln_relu_mm_body(x_ref, lnw_ref, lnb_ref, w_ref, b_ref, o0_ref, o1_ref):
    x = x_ref[...]
    mu = jnp.mean(x, axis=-1, keepdims=True)
    var = jnp.mean((x - mu) * (x - mu), axis=-1, keepdims=True)
    xn = (x - mu) / jnp.sqrt(var + 1e-5) * lnw_ref[...] + lnb_ref[...]
    xn = jnp.maximum(xn, 0.0)
    r = jnp.dot(xn, w_ref[...], preferred_element_type=jnp.float32)
    r = r + b_ref[...]
    o0_ref[...] = r[:, :DH]
    o1_ref[...] = r[:, DH:]


def _final_body(agg_ref, den0_ref, den1_ref, prev_ref, ob_ref,
                lnw_ref, lnb_ref, wm_ref, bm_ref, o_ref):
    den = den0_ref[...] + den1_ref[...]
    d = agg_ref.shape[-1]
    r = lax.broadcasted_iota(jnp.int32, (16, d), 0)
    c = lax.broadcasted_iota(jnp.int32, (16, d), 1)
    expand = (r == c // C).astype(jnp.float32)
    den_bc = jnp.dot(den, expand, preferred_element_type=jnp.float32)
    agg = agg_ref[...] / (den_bc + 1e-16)
    x = prev_ref[...] + agg + ob_ref[...]
    mu = jnp.mean(x, axis=-1, keepdims=True)
    var = jnp.mean((x - mu) * (x - mu), axis=-1, keepdims=True)
    y = (x - mu) / jnp.sqrt(var + 1e-5) * lnw_ref[...] + lnb_ref[...]
    y = jnp.maximum(y, 0.0)
    o_ref[...] = x + (
        jnp.dot(y, wm_ref[...], preferred_element_type=jnp.float32) + bm_ref[...]
    )


def _edge_body(rps, gpw,
               hl0_hbm, hl1_hbm, hr0_hbm, hr1_hbm, src_hbm, dst_hbm,
               att_hbm, zagg_hbm, zden_hbm,
               agg_out, den_out,
               src_v0, src_v1, dst_v0, dst_v1, hl_v, hr_v, ex_v, att_v,
               agg_sp, den_sp, sem):
    cid = _axis_index("c")
    sid = _axis_index("s")
    chunks = gpw // KSUB
    src_vs = (src_v0, src_v1)
    dst_vs = (dst_v0, dst_v1)
    hl_hbms = (hl0_hbm, hl1_hbm)
    hr_hbms = (hr0_hbm, hr1_hbm)

    pltpu.sync_copy(zagg_hbm, agg_sp.at[pl.ds(sid * rps, rps)])
    pltpu.sync_copy(zden_hbm, den_sp.at[pl.ds(sid * rps, rps)])
    pltpu.sync_copy(att_hbm.at[pl.ds(cid * DH, DH)], att_v)
    plsc.subcore_barrier()
    iota16 = lax.iota(jnp.int32, 16)

    def chunk_body(ch, carry):
        g0 = ch * KSUB
        for j in range(KSUB):
            pltpu.sync_copy(src_hbm.at[sid, g0 + j], src_vs[j])
            pltpu.sync_copy(dst_hbm.at[sid, g0 + j], dst_vs[j])
        for variant in range(NC):
            @pl.when(cid == variant)
            def _():
                descs = []
                for j in range(KSUB):
                    descs.append(pltpu.async_copy(
                        hl_hbms[variant].at[src_vs[j]],
                        hl_v.at[pl.ds(j * GATHER, GATHER)], sem))
                    descs.append(pltpu.async_copy(
                        hr_hbms[variant].at[dst_vs[j]],
                        hr_v.at[pl.ds(j * GATHER, GATHER)], sem))
                for dsc in descs:
                    dsc.wait()

        def edge_body(e, c2):
            exrow = jnp.zeros((16,), jnp.float32)
            for lh in range(HL):
                hs = pl.ds(lh * C, C)
                t = hl_v[e, hs] + hr_v[e, hs]
                t = jnp.maximum(t, t * 0.2)
                s = jnp.sum(t * att_v[hs])
                ev = jnp.exp(jnp.broadcast_to(s, (16,)))
                hl_v[e, hs] = hl_v[e, hs] * ev
                exrow = jnp.where(iota16 == cid * HL + lh, ev, exrow)
            ex_v[e, :] = exrow
            return c2

        lax.fori_loop(0, CE, edge_body, 0)

        for j in range(KSUB):
            pltpu.sync_copy(hl_v.at[pl.ds(j * GATHER, GATHER)],
                            agg_sp.at[dst_vs[j]], add=True)
            pltpu.sync_copy(ex_v.at[pl.ds(j * GATHER, GATHER)],
                            den_sp.at[dst_vs[j]], add=True)
        return carry

    lax.fori_loop(0, chunks, chunk_body, 0)
    plsc.subcore_barrier()

    sl = pl.ds(sid * rps, rps)
    pltpu.sync_copy(agg_sp.at[sl], agg_out.at[cid, sl])
    pltpu.sync_copy(den_sp.at[sl], den_out.at[cid, sl])


def _edge_stage(hl0, hl1, hr0, hr1, edge_src, edge_dst, att):
    n_proj = hl0.shape[0]
    n_sp = hr0.shape[0]
    e = edge_src.shape[0]
    e_per_w = e // NS
    rps = -(-(-(-n_sp // NS)) // 8) * 8
    n_pad = NS * rps
    gpw = -(-e_per_w // CE) * KSUB
    pad = gpw * GATHER - e_per_w

    src_w = edge_src.reshape(NS, e_per_w)
    dst_w = edge_dst.reshape(NS, e_per_w)
    srcp = jnp.pad(src_w, ((0, 0), (0, pad))).reshape(NS, gpw, GATHER)
    dstp = jnp.pad(dst_w, ((0, 0), (0, pad)),
                   constant_values=n_sp).reshape(NS, gpw, GATHER)
    zpad = jnp.zeros((PAD, DH), jnp.float32)
    hr0p = jnp.concatenate([hr0, zpad], axis=0)
    hr1p = jnp.concatenate([hr1, zpad], axis=0)
    att_flat = att.reshape(H * C)
    zagg = jnp.zeros((rps, DH), jnp.float32)
    zden = jnp.zeros((rps, 16), jnp.float32)

    mesh = plsc.VectorSubcoreMesh(
        core_axis_name="c", subcore_axis_name="s",
        num_cores=NC, num_subcores=NS)
    fn = pl.kernel(
        functools.partial(_edge_body, rps, gpw),
        out_type=[
            jax.ShapeDtypeStruct((NC, n_pad, DH), jnp.float32),
            jax.ShapeDtypeStruct((NC, n_pad, 16), jnp.float32),
        ],
        mesh=mesh,
        compiler_params=pltpu.CompilerParams(
            needs_layout_passes=False, use_tc_tiling_on_sc=False),
        scratch_types=[
            pltpu.VMEM((GATHER,), jnp.int32),
            pltpu.VMEM((GATHER,), jnp.int32),
            pltpu.VMEM((GATHER,), jnp.int32),
            pltpu.VMEM((GATHER,), jnp.int32),
            pltpu.VMEM((CE, DH), jnp.float32),
            pltpu.VMEM((CE, DH), jnp.float32),
            pltpu.VMEM((CE, 16), jnp.float32),
            pltpu.VMEM((DH,), jnp.float32),
            pltpu.VMEM_SHARED((n_pad, DH), jnp.float32),
            pltpu.VMEM_SHARED((n_pad, 16), jnp.float32),
            pltpu.SemaphoreType.DMA,
        ],
    )
    return fn(hl0, hl1, hr0p, hr1p, srcp, dstp, att_flat, zagg, zden)


def kernel(proj_features, prev_scenepoint_features, edge_src, edge_dst,
           ln1_w, ln1_b, W_l, b_l, W_r, b_r, att, out_bias,
           ln2_w, ln2_b, W_mlp, b_mlp):
    n_proj, d = proj_features.shape
    n_sp = prev_scenepoint_features.shape[0]
    blk_l = 1000
    blk_r = 400

    hl0, hl1 = pl.pallas_call(
        _mm_bias_body,
        grid=(n_proj // blk_l,),
        in_specs=[
            pl.BlockSpec((blk_l, d), lambda i: (i, 0)),
            pl.BlockSpec((d, d), lambda i: (0, 0)),
            pl.BlockSpec((1, d), lambda i: (0, 0)),
        ],
        out_specs=[
            pl.BlockSpec((blk_l, DH), lambda i: (i, 0)),
            pl.BlockSpec((blk_l, DH), lambda i: (i, 0)),
        ],
        out_shape=[
            jax.ShapeDtypeStruct((n_proj, DH), jnp.float32),
            jax.ShapeDtypeStruct((n_proj, DH), jnp.float32),
        ],
    )(proj_features, W_l, b_l[None])

    hr0, hr1 = pl.pallas_call(
        _ln_relu_mm_body,
        grid=(n_sp // blk_r,),
        in_specs=[
            pl.BlockSpec((blk_r, d), lambda i: (i, 0)),
            pl.BlockSpec((1, d), lambda i: (0, 0)),
            pl.BlockSpec((1, d), lambda i: (0, 0)),
            pl.BlockSpec((d, d), lambda i: (0, 0)),
            pl.BlockSpec((1, d), lambda i: (0, 0)),
        ],
        out_specs=[
            pl.BlockSpec((blk_r, DH), lambda i: (i, 0)),
            pl.BlockSpec((blk_r, DH), lambda i: (i, 0)),
        ],
        out_shape=[
            jax.ShapeDtypeStruct((n_sp, DH), jnp.float32),
            jax.ShapeDtypeStruct((n_sp, DH), jnp.float32),
        ],
    )(prev_scenepoint_features, ln1_w[None], ln1_b[None], W_r, b_r[None])

    agg, den = _edge_stage(hl0, hl1, hr0, hr1, edge_src, edge_dst, att)
    agg_cat = jnp.concatenate([agg[0], agg[1]], axis=1)

    out = pl.pallas_call(
        _final_body,
        grid=(n_sp // blk_r,),
        in_specs=[
            pl.BlockSpec((blk_r, d), lambda i: (i, 0)),
            pl.BlockSpec((blk_r, 16), lambda i: (i, 0)),
            pl.BlockSpec((blk_r, 16), lambda i: (i, 0)),
            pl.BlockSpec((blk_r, d), lambda i: (i, 0)),
            pl.BlockSpec((1, d), lambda i: (0, 0)),
            pl.BlockSpec((1, d), lambda i: (0, 0)),
            pl.BlockSpec((1, d), lambda i: (0, 0)),
            pl.BlockSpec((d, d), lambda i: (0, 0)),
            pl.BlockSpec((1, d), lambda i: (0, 0)),
        ],
        out_specs=pl.BlockSpec((blk_r, d), lambda i: (i, 0)),
        out_shape=jax.ShapeDtypeStruct((n_sp, d), jnp.float32),
    )(agg_cat, den[0], den[1], prev_scenepoint_features,
      out_bias[None], ln2_w[None], ln2_b[None], W_mlp, b_mlp[None])
    return out


# speedup vs baseline: 63.5191x; 2.8756x over previous
import functools

import jax
import jax.numpy as jnp
from jax import lax
from jax.experimental import pallas as pl
from jax.experimental.pallas import tpu as pltpu
from jax.experimental.pallas import tpu_sc as plsc

NC = 2
NS = 16
H = 8
HL = H // NC
C = 16
DH = HL * C
GATHER = 128
KSUB = 2
CE = GATHER * KSUB
PAD = 16


def _axis_index(name):
    return lax.axis_index(name)


def _mm_bias_body(x_ref, w_ref, b_ref, o0_ref, o1_ref):
    r = jnp.dot(x_ref[...], w_ref[...], preferred_element_type=jnp.float32)
    r = r + b_ref[...]
    o0_ref[...] = r[:, :DH]
    o1_ref[...] = r[:, DH:]


def _ln_relu_mm_body(x_ref, lnw_ref, lnb_ref, w_ref, b_ref, o0_ref, o1_ref):
    x = x_ref[...]
    mu = jnp.mean(x, axis=-1, keepdims=True)
    var = jnp.mean((x - mu) * (x - mu), axis=-1, keepdims=True)
    xn = (x - mu) / jnp.sqrt(var + 1e-5) * lnw_ref[...] + lnb_ref[...]
    xn = jnp.maximum(xn, 0.0)
    r = jnp.dot(xn, w_ref[...], preferred_element_type=jnp.float32)
    r = r + b_ref[...]
    o0_ref[...] = r[:, :DH]
    o1_ref[...] = r[:, DH:]


def _final_body(agg_ref, den0_ref, den1_ref, prev_ref, ob_ref,
                lnw_ref, lnb_ref, wm_ref, bm_ref, o_ref):
    den = den0_ref[...] + den1_ref[...]
    d = agg_ref.shape[-1]
    r = lax.broadcasted_iota(jnp.int32, (16, d), 0)
    c = lax.broadcasted_iota(jnp.int32, (16, d), 1)
    expand = (r == c // C).astype(jnp.float32)
    den_bc = jnp.dot(den, expand, preferred_element_type=jnp.float32)
    agg = agg_ref[...] / (den_bc + 1e-16)
    x = prev_ref[...] + agg + ob_ref[...]
    mu = jnp.mean(x, axis=-1, keepdims=True)
    var = jnp.mean((x - mu) * (x - mu), axis=-1, keepdims=True)
    y = (x - mu) / jnp.sqrt(var + 1e-5) * lnw_ref[...] + lnb_ref[...]
    y = jnp.maximum(y, 0.0)
    o_ref[...] = x + (
        jnp.dot(y, wm_ref[...], preferred_element_type=jnp.float32) + bm_ref[...]
    )


def _edge_body(rps, gpw,
               hl0_hbm, hl1_hbm, hr0_hbm, hr1_hbm, src_hbm, dst_hbm,
               att_hbm, zagg_hbm, zden_hbm,
               agg_out, den_out,
               s0a, s0b, d0a, d0b, s1a, s1b, d1a, d1b,
               hl_a, hl_b, hr_a, hr_b, ex_v, att_v,
               agg_sp, den_sp, sem_a, sem_b):
    cid = _axis_index("c")
    sid = _axis_index("s")
    chunks = gpw // KSUB
    sidx = ((s0a, s0b), (s1a, s1b))
    didx = ((d0a, d0b), (d1a, d1b))
    hls = (hl_a, hl_b)
    hrs = (hr_a, hr_b)
    sems = (sem_a, sem_b)
    hl_hbms = (hl0_hbm, hl1_hbm)
    hr_hbms = (hr0_hbm, hr1_hbm)

    pltpu.sync_copy(zagg_hbm, agg_sp.at[pl.ds(sid * rps, rps)])
    pltpu.sync_copy(zden_hbm, den_sp.at[pl.ds(sid * rps, rps)])
    pltpu.sync_copy(att_hbm.at[pl.ds(cid * DH, DH)], att_v)
    plsc.subcore_barrier()
    iota16 = lax.iota(jnp.int32, 16)
    attv = tuple(att_v[pl.ds(lh * C, C)] for lh in range(HL))

    def issue(ch, b):
        g0 = ch * KSUB
        for j in range(KSUB):
            pltpu.sync_copy(src_hbm.at[sid, g0 + j], sidx[b][j])
            pltpu.sync_copy(dst_hbm.at[sid, g0 + j], didx[b][j])
        for variant in range(NC):
            @pl.when(cid == variant)
            def _():
                for j in range(KSUB):
                    pltpu.async_copy(
                        hl_hbms[variant].at[sidx[b][j]],
                        hls[b].at[pl.ds(j * GATHER, GATHER)], sems[b])
                    pltpu.async_copy(
                        hr_hbms[variant].at[didx[b][j]],
                        hrs[b].at[pl.ds(j * GATHER, GATHER)], sems[b])

    def drain(b):
        for j in range(KSUB):
            pltpu.make_async_copy(
                hl_hbms[0].at[sidx[b][j]],
                hls[b].at[pl.ds(j * GATHER, GATHER)], sems[b]).wait()
            pltpu.make_async_copy(
                hr_hbms[0].at[didx[b][j]],
                hrs[b].at[pl.ds(j * GATHER, GATHER)], sems[b]).wait()

    def compute_scatter(b):
        @plsc.parallel_loop(0, CE, unroll=8)
        def edge_body(e):
            exrow = jnp.zeros((16,), jnp.float32)
            for lh in range(HL):
                hs = pl.ds(lh * C, C)
                a = hls[b][e, hs]
                t = a + hrs[b][e, hs]
                t = jnp.maximum(t, t * 0.2)
                s = jnp.sum(t * attv[lh])
                ev = jnp.exp(jnp.broadcast_to(s, (16,)))
                hls[b][e, hs] = a * ev
                exrow = jnp.where(iota16 == cid * HL + lh, ev, exrow)
            ex_v[e, :] = exrow

        for j in range(KSUB):
            pltpu.sync_copy(hls[b].at[pl.ds(j * GATHER, GATHER)],
                            agg_sp.at[didx[b][j]], add=True)
            pltpu.sync_copy(ex_v.at[pl.ds(j * GATHER, GATHER)],
                            den_sp.at[didx[b][j]], add=True)

    issue(0, 0)

    def outer_body(cc, carry):
        for phase in range(2):
            b = phase
            ch = cc * 2 + phase

            @pl.when(ch + 1 < chunks)
            def _():
                issue(ch + 1, 1 - b)

            drain(b)
            compute_scatter(b)
        return carry

    lax.fori_loop(0, chunks // 2, outer_body, 0)
    plsc.subcore_barrier()

    sl = pl.ds(sid * rps, rps)
    pltpu.sync_copy(agg_sp.at[sl], agg_out.at[cid, sl])
    pltpu.sync_copy(den_sp.at[sl], den_out.at[cid, sl])


def _edge_stage(hl0, hl1, hr0, hr1, edge_src, edge_dst, att):
    n_proj = hl0.shape[0]
    n_sp = hr0.shape[0]
    e = edge_src.shape[0]
    e_per_w = e // NS
    rps = -(-(-(-n_sp // NS)) // 8) * 8
    n_pad = NS * rps
    gpw = -(-e_per_w // (2 * CE)) * 2 * KSUB
    pad = gpw * GATHER - e_per_w

    src_w = edge_src.reshape(NS, e_per_w)
    dst_w = edge_dst.reshape(NS, e_per_w)
    srcp = jnp.pad(src_w, ((0, 0), (0, pad))).reshape(NS, gpw, GATHER)
    dstp = jnp.pad(dst_w, ((0, 0), (0, pad)),
                   constant_values=n_sp).reshape(NS, gpw, GATHER)
    zpad = jnp.zeros((PAD, DH), jnp.float32)
    hr0p = jnp.concatenate([hr0, zpad], axis=0)
    hr1p = jnp.concatenate([hr1, zpad], axis=0)
    att_flat = att.reshape(H * C)
    zagg = jnp.zeros((rps, DH), jnp.float32)
    zden = jnp.zeros((rps, 16), jnp.float32)

    mesh = plsc.VectorSubcoreMesh(
        core_axis_name="c", subcore_axis_name="s",
        num_cores=NC, num_subcores=NS)
    fn = pl.kernel(
        functools.partial(_edge_body, rps, gpw),
        out_type=[
            jax.ShapeDtypeStruct((NC, n_pad, DH), jnp.float32),
            jax.ShapeDtypeStruct((NC, n_pad, 16), jnp.float32),
        ],
        mesh=mesh,
        compiler_params=pltpu.CompilerParams(
            needs_layout_passes=False, use_tc_tiling_on_sc=False),
        scratch_types=(
            [pltpu.VMEM((GATHER,), jnp.int32)] * 8 +
            [
                pltpu.VMEM((CE, DH), jnp.float32),
                pltpu.VMEM((CE, DH), jnp.float32),
                pltpu.VMEM((CE, DH), jnp.float32),
                pltpu.VMEM((CE, DH), jnp.float32),
                pltpu.VMEM((CE, 16), jnp.float32),
                pltpu.VMEM((DH,), jnp.float32),
                pltpu.VMEM_SHARED((n_pad, DH), jnp.float32),
                pltpu.VMEM_SHARED((n_pad, 16), jnp.float32),
                pltpu.SemaphoreType.DMA,
                pltpu.SemaphoreType.DMA,
            ]
        ),
    )
    return fn(hl0, hl1, hr0p, hr1p, srcp, dstp, att_flat, zagg, zden)


def kernel(proj_features, prev_scenepoint_features, edge_src, edge_dst,
           ln1_w, ln1_b, W_l, b_l, W_r, b_r, att, out_bias,
           ln2_w, ln2_b, W_mlp, b_mlp):
    n_proj, d = proj_features.shape
    n_sp = prev_scenepoint_features.shape[0]
    blk_l = 1000
    blk_r = 400

    hl0, hl1 = pl.pallas_call(
        _mm_bias_body,
        grid=(n_proj // blk_l,),
        in_specs=[
            pl.BlockSpec((blk_l, d), lambda i: (i, 0)),
            pl.BlockSpec((d, d), lambda i: (0, 0)),
            pl.BlockSpec((1, d), lambda i: (0, 0)),
        ],
        out_specs=[
            pl.BlockSpec((blk_l, DH), lambda i: (i, 0)),
            pl.BlockSpec((blk_l, DH), lambda i: (i, 0)),
        ],
        out_shape=[
            jax.ShapeDtypeStruct((n_proj, DH), jnp.float32),
            jax.ShapeDtypeStruct((n_proj, DH), jnp.float32),
        ],
    )(proj_features, W_l, b_l[None])

    hr0, hr1 = pl.pallas_call(
        _ln_relu_mm_body,
        grid=(n_sp // blk_r,),
        in_specs=[
            pl.BlockSpec((blk_r, d), lambda i: (i, 0)),
            pl.BlockSpec((1, d), lambda i: (0, 0)),
            pl.BlockSpec((1, d), lambda i: (0, 0)),
            pl.BlockSpec((d, d), lambda i: (0, 0)),
            pl.BlockSpec((1, d), lambda i: (0, 0)),
        ],
        out_specs=[
            pl.BlockSpec((blk_r, DH), lambda i: (i, 0)),
            pl.BlockSpec((blk_r, DH), lambda i: (i, 0)),
        ],
        out_shape=[
            jax.ShapeDtypeStruct((n_sp, DH), jnp.float32),
            jax.ShapeDtypeStruct((n_sp, DH), jnp.float32),
        ],
    )(prev_scenepoint_features, ln1_w[None], ln1_b[None], W_r, b_r[None])

    agg, den = _edge_stage(hl0, hl1, hr0, hr1, edge_src, edge_dst, att)
    agg_cat = jnp.concatenate([agg[0], agg[1]], axis=1)

    out = pl.pallas_call(
        _final_body,
        grid=(n_sp // blk_r,),
        in_specs=[
            pl.BlockSpec((blk_r, d), lambda i: (i, 0)),
            pl.BlockSpec((blk_r, 16), lambda i: (i, 0)),
            pl.BlockSpec((blk_r, 16), lambda i: (i, 0)),
            pl.BlockSpec((blk_r, d), lambda i: (i, 0)),
            pl.BlockSpec((1, d), lambda i: (0, 0)),
            pl.BlockSpec((1, d), lambda i: (0, 0)),
            pl.BlockSpec((1, d), lambda i: (0, 0)),
            pl.BlockSpec((d, d), lambda i: (0, 0)),
            pl.BlockSpec((1, d), lambda i: (0, 0)),
        ],
        out_specs=pl.BlockSpec((blk_r, d), lambda i: (i, 0)),
        out_shape=jax.ShapeDtypeStruct((n_sp, d), jnp.float32),
    )(agg_cat, den[0], den[1], prev_scenepoint_features,
      out_bias[None], ln2_w[None], ln2_b[None], W_mlp, b_mlp[None])
    return out


# speedup vs baseline: 68.6126x; 1.0802x over previous
import functools

import jax
import jax.numpy as jnp
from jax import lax
from jax.experimental import pallas as pl
from jax.experimental.pallas import tpu as pltpu
from jax.experimental.pallas import tpu_sc as plsc

NC = 2
NS = 16
H = 8
HL = H // NC
C = 16
DH = HL * C
GATHER = 128
KSUB = 2
CE = GATHER * KSUB
PAD = 16


def _axis_index(name):
    return lax.axis_index(name)


def _pre_body(proj_ref, prev_ref, wl_ref, bl_ref, lnw_ref, lnb_ref,
              wr_ref, br_ref, hl0_ref, hl1_ref, hr0_ref, hr1_ref):
    r = jnp.dot(proj_ref[...], wl_ref[...], preferred_element_type=jnp.float32)
    r = r + bl_ref[...]
    hl0_ref[...] = r[:, :DH]
    hl1_ref[...] = r[:, DH:]
    x = prev_ref[...]
    mu = jnp.mean(x, axis=-1, keepdims=True)
    var = jnp.mean((x - mu) * (x - mu), axis=-1, keepdims=True)
    xn = (x - mu) / jnp.sqrt(var + 1e-5) * lnw_ref[...] + lnb_ref[...]
    xn = jnp.maximum(xn, 0.0)
    rr = jnp.dot(xn, wr_ref[...], preferred_element_type=jnp.float32)
    rr = rr + br_ref[...]
    hr0_ref[...] = rr[:, :DH]
    hr1_ref[...] = rr[:, DH:]


def _final_body(agg0_ref, agg1_ref, den0_ref, den1_ref, prev_ref, ob_ref,
                lnw_ref, lnb_ref, wm_ref, bm_ref, o_ref):
    den = den0_ref[...] + den1_ref[...]
    agg_full = jnp.concatenate([agg0_ref[...], agg1_ref[...]], axis=1)
    d = agg_full.shape[-1]
    r = lax.broadcasted_iota(jnp.int32, (16, d), 0)
    c = lax.broadcasted_iota(jnp.int32, (16, d), 1)
    expand = (r == c // C).astype(jnp.float32)
    den_bc = jnp.dot(den, expand, preferred_element_type=jnp.float32)
    agg = agg_full / (den_bc + 1e-16)
    x = prev_ref[...] + agg + ob_ref[...]
    mu = jnp.mean(x, axis=-1, keepdims=True)
    var = jnp.mean((x - mu) * (x - mu), axis=-1, keepdims=True)
    y = (x - mu) / jnp.sqrt(var + 1e-5) * lnw_ref[...] + lnb_ref[...]
    y = jnp.maximum(y, 0.0)
    o_ref[...] = x + (
        jnp.dot(y, wm_ref[...], preferred_element_type=jnp.float32) + bm_ref[...]
    )


def _edge_body(rps, gpw,
               hl0_hbm, hl1_hbm, hr0_hbm, hr1_hbm, src_hbm, dst_hbm,
               att_hbm, zagg_hbm, zden_hbm,
               agg_out, den_out,
               s0a, s0b, d0a, d0b, s1a, s1b, d1a, d1b,
               hl_a, hl_b, hr_a, hr_b, ex_v, att_v,
               agg_sp, den_sp, sem_a, sem_b):
    cid = _axis_index("c")
    sid = _axis_index("s")
    chunks = gpw // KSUB
    sidx = ((s0a, s0b), (s1a, s1b))
    didx = ((d0a, d0b), (d1a, d1b))
    hls = (hl_a, hl_b)
    hrs = (hr_a, hr_b)
    sems = (sem_a, sem_b)
    hl_hbms = (hl0_hbm, hl1_hbm)
    hr_hbms = (hr0_hbm, hr1_hbm)

    pltpu.sync_copy(zagg_hbm, agg_sp.at[pl.ds(sid * rps, rps)])
    pltpu.sync_copy(zden_hbm, den_sp.at[pl.ds(sid * rps, rps)])
    pltpu.sync_copy(att_hbm.at[pl.ds(cid * DH, DH)], att_v)
    plsc.subcore_barrier()
    iota16 = lax.iota(jnp.int32, 16)
    attv = tuple(att_v[pl.ds(lh * C, C)] for lh in range(HL))

    def issue(ch, b):
        g0 = ch * KSUB
        for j in range(KSUB):
            pltpu.sync_copy(src_hbm.at[sid, g0 + j], sidx[b][j])
            pltpu.sync_copy(dst_hbm.at[sid, g0 + j], didx[b][j])
        for variant in range(NC):
            @pl.when(cid == variant)
            def _():
                for j in range(KSUB):
                    pltpu.async_copy(
                        hl_hbms[variant].at[sidx[b][j]],
                        hls[b].at[pl.ds(j * GATHER, GATHER)], sems[b])
                    pltpu.async_copy(
                        hr_hbms[variant].at[didx[b][j]],
                        hrs[b].at[pl.ds(j * GATHER, GATHER)], sems[b])

    def drain(b):
        for j in range(KSUB):
            pltpu.make_async_copy(
                hl_hbms[0].at[sidx[b][j]],
                hls[b].at[pl.ds(j * GATHER, GATHER)], sems[b]).wait()
            pltpu.make_async_copy(
                hr_hbms[0].at[didx[b][j]],
                hrs[b].at[pl.ds(j * GATHER, GATHER)], sems[b]).wait()

    def compute_scatter(b):
        @plsc.parallel_loop(0, CE, unroll=16)
        def edge_body(e):
            exrow = jnp.zeros((16,), jnp.float32)
            for lh in range(HL):
                hs = pl.ds(lh * C, C)
                a = hls[b][e, hs]
                t = a + hrs[b][e, hs]
                t = jnp.maximum(t, t * 0.2)
                s = jnp.sum(t * attv[lh])
                ev = jnp.exp(jnp.broadcast_to(s, (16,)))
                hls[b][e, hs] = a * ev
                exrow = jnp.where(iota16 == cid * HL + lh, ev, exrow)
            ex_v[e, :] = exrow

        for j in range(KSUB):
            pltpu.sync_copy(hls[b].at[pl.ds(j * GATHER, GATHER)],
                            agg_sp.at[didx[b][j]], add=True)
            pltpu.sync_copy(ex_v.at[pl.ds(j * GATHER, GATHER)],
                            den_sp.at[didx[b][j]], add=True)

    issue(0, 0)

    def outer_body(cc, carry):
        for phase in range(2):
            b = phase
            ch = cc * 2 + phase

            @pl.when(ch + 1 < chunks)
            def _():
                issue(ch + 1, 1 - b)

            drain(b)
            compute_scatter(b)
        return carry

    lax.fori_loop(0, chunks // 2, outer_body, 0)
    plsc.subcore_barrier()

    sl = pl.ds(sid * rps, rps)
    pltpu.sync_copy(agg_sp.at[sl], agg_out.at[cid, sl])
    pltpu.sync_copy(den_sp.at[sl], den_out.at[cid, sl])


def _edge_stage(hl0, hl1, hr0, hr1, edge_src, edge_dst, att):
    n_proj = hl0.shape[0]
    n_sp = hr0.shape[0] - PAD
    e = edge_src.shape[0]
    e_per_w = e // NS
    rps = -(-(-(-n_sp // NS)) // 8) * 8
    n_pad = NS * rps
    gpw = -(-e_per_w // (2 * CE)) * 2 * KSUB
    pad = gpw * GATHER - e_per_w

    src_w = edge_src.reshape(NS, e_per_w)
    dst_w = edge_dst.reshape(NS, e_per_w)
    srcp = jnp.pad(src_w, ((0, 0), (0, pad))).reshape(NS, gpw, GATHER)
    dstp = jnp.pad(dst_w, ((0, 0), (0, pad)),
                   constant_values=n_sp).reshape(NS, gpw, GATHER)
    att_flat = att.reshape(H * C)
    zagg = jnp.zeros((rps, DH), jnp.float32)
    zden = jnp.zeros((rps, 16), jnp.float32)

    mesh = plsc.VectorSubcoreMesh(
        core_axis_name="c", subcore_axis_name="s",
        num_cores=NC, num_subcores=NS)
    fn = pl.kernel(
        functools.partial(_edge_body, rps, gpw),
        out_type=[
            jax.ShapeDtypeStruct((NC, n_pad, DH), jnp.float32),
            jax.ShapeDtypeStruct((NC, n_pad, 16), jnp.float32),
        ],
        mesh=mesh,
        compiler_params=pltpu.CompilerParams(
            needs_layout_passes=False, use_tc_tiling_on_sc=False),
        scratch_types=(
            [pltpu.VMEM((GATHER,), jnp.int32)] * 8 +
            [
                pltpu.VMEM((CE, DH), jnp.float32),
                pltpu.VMEM((CE, DH), jnp.float32),
                pltpu.VMEM((CE, DH), jnp.float32),
                pltpu.VMEM((CE, DH), jnp.float32),
                pltpu.VMEM((CE, 16), jnp.float32),
                pltpu.VMEM((DH,), jnp.float32),
                pltpu.VMEM_SHARED((n_pad, DH), jnp.float32),
                pltpu.VMEM_SHARED((n_pad, 16), jnp.float32),
                pltpu.SemaphoreType.DMA,
                pltpu.SemaphoreType.DMA,
            ]
        ),
    )
    return fn(hl0, hl1, hr0, hr1, srcp, dstp, att_flat, zagg, zden)


def kernel(proj_features, prev_scenepoint_features, edge_src, edge_dst,
           ln1_w, ln1_b, W_l, b_l, W_r, b_r, att, out_bias,
           ln2_w, ln2_b, W_mlp, b_mlp):
    n_proj, d = proj_features.shape
    n_sp = prev_scenepoint_features.shape[0]
    blk_l = 1000
    blk_p = n_sp // (n_proj // blk_l)
    blk_r = 400

    hl0, hl1, hr0, hr1 = pl.pallas_call(
        _pre_body,
        grid=(n_proj // blk_l,),
        in_specs=[
            pl.BlockSpec((blk_l, d), lambda i: (i, 0)),
            pl.BlockSpec((blk_p, d), lambda i: (i, 0)),
            pl.BlockSpec((d, d), lambda i: (0, 0)),
            pl.BlockSpec((1, d), lambda i: (0, 0)),
            pl.BlockSpec((1, d), lambda i: (0, 0)),
            pl.BlockSpec((1, d), lambda i: (0, 0)),
            pl.BlockSpec((d, d), lambda i: (0, 0)),
            pl.BlockSpec((1, d), lambda i: (0, 0)),
        ],
        out_specs=[
            pl.BlockSpec((blk_l, DH), lambda i: (i, 0)),
            pl.BlockSpec((blk_l, DH), lambda i: (i, 0)),
            pl.BlockSpec((blk_p, DH), lambda i: (i, 0)),
            pl.BlockSpec((blk_p, DH), lambda i: (i, 0)),
        ],
        out_shape=[
            jax.ShapeDtypeStruct((n_proj, DH), jnp.float32),
            jax.ShapeDtypeStruct((n_proj, DH), jnp.float32),
            jax.ShapeDtypeStruct((n_sp + PAD, DH), jnp.float32),
            jax.ShapeDtypeStruct((n_sp + PAD, DH), jnp.float32),
        ],
    )(proj_features, prev_scenepoint_features, W_l, b_l[None],
      ln1_w[None], ln1_b[None], W_r, b_r[None])

    agg, den = _edge_stage(hl0, hl1, hr0, hr1, edge_src, edge_dst, att)

    out = pl.pallas_call(
        _final_body,
        grid=(n_sp // blk_r,),
        in_specs=[
            pl.BlockSpec((blk_r, DH), lambda i: (i, 0)),
            pl.BlockSpec((blk_r, DH), lambda i: (i, 0)),
            pl.BlockSpec((blk_r, 16), lambda i: (i, 0)),
            pl.BlockSpec((blk_r, 16), lambda i: (i, 0)),
            pl.BlockSpec((blk_r, d), lambda i: (i, 0)),
            pl.BlockSpec((1, d), lambda i: (0, 0)),
            pl.BlockSpec((1, d), lambda i: (0, 0)),
            pl.BlockSpec((1, d), lambda i: (0, 0)),
            pl.BlockSpec((d, d), lambda i: (0, 0)),
            pl.BlockSpec((1, d), lambda i: (0, 0)),
        ],
        out_specs=pl.BlockSpec((blk_r, d), lambda i: (i, 0)),
        out_shape=jax.ShapeDtypeStruct((n_sp, d), jnp.float32),
    )(agg[0], agg[1], den[0], den[1], prev_scenepoint_features,
      out_bias[None], ln2_w[None], ln2_b[None], W_mlp, b_mlp[None])
    return out


# speedup vs baseline: 70.2047x; 1.0232x over previous
import functools

import jax
import jax.numpy as jnp
from jax import lax
from jax.experimental import pallas as pl
from jax.experimental.pallas import tpu as pltpu
from jax.experimental.pallas import tpu_sc as plsc

NC = 2
NS = 16
H = 8
HL = H // NC
C = 16
DH = HL * C
GATHER = 128
KSUB = 2
CE = GATHER * KSUB
PAD = 16


def _axis_index(name):
    return lax.axis_index(name)


def _pre_body(proj_ref, prev_ref, wl_ref, bl_ref, lnw_ref, lnb_ref,
              wr_ref, br_ref, hl0_ref, hl1_ref, hr0_ref, hr1_ref):
    r = jnp.dot(proj_ref[...], wl_ref[...], preferred_element_type=jnp.float32)
    r = r + bl_ref[...]
    hl0_ref[...] = r[:, :DH]
    hl1_ref[...] = r[:, DH:]
    x = prev_ref[...]
    mu = jnp.mean(x, axis=-1, keepdims=True)
    var = jnp.mean((x - mu) * (x - mu), axis=-1, keepdims=True)
    xn = (x - mu) / jnp.sqrt(var + 1e-5) * lnw_ref[...] + lnb_ref[...]
    xn = jnp.maximum(xn, 0.0)
    rr = jnp.dot(xn, wr_ref[...], preferred_element_type=jnp.float32)
    rr = rr + br_ref[...]
    hr0_ref[...] = rr[:, :DH]
    hr1_ref[...] = rr[:, DH:]


def _final_body(agg0_ref, agg1_ref, den0_ref, den1_ref, prev_ref, ob_ref,
                lnw_ref, lnb_ref, wm_ref, bm_ref, o_ref):
    den = den0_ref[...] + den1_ref[...]
    agg_full = jnp.concatenate([agg0_ref[...], agg1_ref[...]], axis=1)
    d = agg_full.shape[-1]
    r = lax.broadcasted_iota(jnp.int32, (16, d), 0)
    c = lax.broadcasted_iota(jnp.int32, (16, d), 1)
    expand = (r == c // C).astype(jnp.float32)
    den_bc = jnp.dot(den, expand, preferred_element_type=jnp.float32)
    agg = agg_full / (den_bc + 1e-16)
    x = prev_ref[...] + agg + ob_ref[...]
    mu = jnp.mean(x, axis=-1, keepdims=True)
    var = jnp.mean((x - mu) * (x - mu), axis=-1, keepdims=True)
    y = (x - mu) / jnp.sqrt(var + 1e-5) * lnw_ref[...] + lnb_ref[...]
    y = jnp.maximum(y, 0.0)
    o_ref[...] = x + (
        jnp.dot(y, wm_ref[...], preferred_element_type=jnp.float32) + bm_ref[...]
    )


def _edge_body(rps, gpw,
               hl0_hbm, hl1_hbm, hr0_hbm, hr1_hbm, src_hbm, dst_hbm,
               att_hbm, zagg_hbm, zden_hbm,
               agg_out, den_out,
               s0a, s0b, d0a, d0b, s1a, s1b, d1a, d1b,
               hl_a, hl_b, hr_a, hr_b, ex_a, ex_b, att_v,
               agg_sp, den_sp, sem_a, sem_b, ssem_a, ssem_b):
    cid = _axis_index("c")
    sid = _axis_index("s")
    chunks = gpw // KSUB
    sidx = ((s0a, s0b), (s1a, s1b))
    didx = ((d0a, d0b), (d1a, d1b))
    hls = (hl_a, hl_b)
    hrs = (hr_a, hr_b)
    exs = (ex_a, ex_b)
    sems = (sem_a, sem_b)
    ssems = (ssem_a, ssem_b)
    hl_hbms = (hl0_hbm, hl1_hbm)
    hr_hbms = (hr0_hbm, hr1_hbm)

    pltpu.sync_copy(zagg_hbm, agg_sp.at[pl.ds(sid * rps, rps)])
    pltpu.sync_copy(zden_hbm, den_sp.at[pl.ds(sid * rps, rps)])
    pltpu.sync_copy(att_hbm.at[pl.ds(cid * DH, DH)], att_v)
    plsc.subcore_barrier()
    iota16 = lax.iota(jnp.int32, 16)
    attv = tuple(att_v[pl.ds(lh * C, C)] for lh in range(HL))

    def issue(ch, b):
        g0 = ch * KSUB
        for j in range(KSUB):
            pltpu.sync_copy(src_hbm.at[sid, g0 + j], sidx[b][j])
            pltpu.sync_copy(dst_hbm.at[sid, g0 + j], didx[b][j])
        for variant in range(NC):
            @pl.when(cid == variant)
            def _():
                for j in range(KSUB):
                    pltpu.async_copy(
                        hl_hbms[variant].at[sidx[b][j]],
                        hls[b].at[pl.ds(j * GATHER, GATHER)], sems[b])
                    pltpu.async_copy(
                        hr_hbms[variant].at[didx[b][j]],
                        hrs[b].at[pl.ds(j * GATHER, GATHER)], sems[b])

    def drain(b):
        for j in range(KSUB):
            pltpu.make_async_copy(
                hl_hbms[0].at[sidx[b][j]],
                hls[b].at[pl.ds(j * GATHER, GATHER)], sems[b]).wait()
            pltpu.make_async_copy(
                hr_hbms[0].at[didx[b][j]],
                hrs[b].at[pl.ds(j * GATHER, GATHER)], sems[b]).wait()

    def compute(b):
        @plsc.parallel_loop(0, CE, unroll=16)
        def edge_body(e):
            exrow = jnp.zeros((16,), jnp.float32)
            for lh in range(HL):
                hs = pl.ds(lh * C, C)
                a = hls[b][e, hs]
                t = a + hrs[b][e, hs]
                t = jnp.maximum(t, t * 0.2)
                s = jnp.sum(t * attv[lh])
                ev = jnp.exp(jnp.broadcast_to(s, (16,)))
                hls[b][e, hs] = a * ev
                exrow = jnp.where(iota16 == cid * HL + lh, ev, exrow)
            exs[b][e, :] = exrow

    def scatter_issue(b):
        for j in range(KSUB):
            pltpu.async_copy(hls[b].at[pl.ds(j * GATHER, GATHER)],
                             agg_sp.at[didx[b][j]], ssems[b], add=True)
            pltpu.async_copy(exs[b].at[pl.ds(j * GATHER, GATHER)],
                             den_sp.at[didx[b][j]], ssems[b], add=True)

    def scatter_drain(b):
        for j in range(KSUB):
            pltpu.make_async_copy(hls[b].at[pl.ds(j * GATHER, GATHER)],
                                  agg_sp.at[didx[b][j]], ssems[b]).wait()
            pltpu.make_async_copy(exs[b].at[pl.ds(j * GATHER, GATHER)],
                                  den_sp.at[didx[b][j]], ssems[b]).wait()

    issue(0, 0)

    def outer_body(cc, carry):
        for phase in range(2):
            b = phase
            ch = cc * 2 + phase

            @pl.when(ch >= 1)
            def _():
                scatter_drain(1 - b)

            @pl.when(ch + 1 < chunks)
            def _():
                issue(ch + 1, 1 - b)

            drain(b)
            compute(b)
            scatter_issue(b)
        return carry

    lax.fori_loop(0, chunks // 2, outer_body, 0)
    scatter_drain(1)
    plsc.subcore_barrier()

    sl = pl.ds(sid * rps, rps)
    pltpu.sync_copy(agg_sp.at[sl], agg_out.at[cid, sl])
    pltpu.sync_copy(den_sp.at[sl], den_out.at[cid, sl])


def _edge_stage(hl0, hl1, hr0, hr1, edge_src, edge_dst, att):
    n_proj = hl0.shape[0]
    n_sp = hr0.shape[0] - PAD
    e = edge_src.shape[0]
    e_per_w = e // NS
    rps = -(-(-(-n_sp // NS)) // 8) * 8
    n_pad = NS * rps
    gpw = -(-e_per_w // (2 * CE)) * 2 * KSUB
    pad = gpw * GATHER - e_per_w

    src_w = edge_src.reshape(NS, e_per_w)
    dst_w = edge_dst.reshape(NS, e_per_w)
    srcp = jnp.pad(src_w, ((0, 0), (0, pad))).reshape(NS, gpw, GATHER)
    dstp = jnp.pad(dst_w, ((0, 0), (0, pad)),
                   constant_values=n_sp).reshape(NS, gpw, GATHER)
    att_flat = att.reshape(H * C)
    zagg = jnp.zeros((rps, DH), jnp.float32)
    zden = jnp.zeros((rps, 16), jnp.float32)

    mesh = plsc.VectorSubcoreMesh(
        core_axis_name="c", subcore_axis_name="s",
        num_cores=NC, num_subcores=NS)
    fn = pl.kernel(
        functools.partial(_edge_body, rps, gpw),
        out_type=[
            jax.ShapeDtypeStruct((NC, n_pad, DH), jnp.float32),
            jax.ShapeDtypeStruct((NC, n_pad, 16), jnp.float32),
        ],
        mesh=mesh,
        compiler_params=pltpu.CompilerParams(
            needs_layout_passes=False, use_tc_tiling_on_sc=False),
        scratch_types=(
            [pltpu.VMEM((GATHER,), jnp.int32)] * 8 +
            [
                pltpu.VMEM((CE, DH), jnp.float32),
                pltpu.VMEM((CE, DH), jnp.float32),
                pltpu.VMEM((CE, DH), jnp.float32),
                pltpu.VMEM((CE, DH), jnp.float32),
                pltpu.VMEM((CE, 16), jnp.float32),
                pltpu.VMEM((CE, 16), jnp.float32),
                pltpu.VMEM((DH,), jnp.float32),
                pltpu.VMEM_SHARED((n_pad, DH), jnp.float32),
                pltpu.VMEM_SHARED((n_pad, 16), jnp.float32),
                pltpu.SemaphoreType.DMA,
                pltpu.SemaphoreType.DMA,
                pltpu.SemaphoreType.DMA,
                pltpu.SemaphoreType.DMA,
            ]
        ),
    )
    return fn(hl0, hl1, hr0, hr1, srcp, dstp, att_flat, zagg, zden)


def kernel(proj_features, prev_scenepoint_features, edge_src, edge_dst,
           ln1_w, ln1_b, W_l, b_l, W_r, b_r, att, out_bias,
           ln2_w, ln2_b, W_mlp, b_mlp):
    n_proj, d = proj_features.shape
    n_sp = prev_scenepoint_features.shape[0]
    blk_l = 1000
    blk_p = n_sp // (n_proj // blk_l)
    blk_r = 400

    hl0, hl1, hr0, hr1 = pl.pallas_call(
        _pre_body,
        grid=(n_proj // blk_l,),
        in_specs=[
            pl.BlockSpec((blk_l, d), lambda i: (i, 0)),
            pl.BlockSpec((blk_p, d), lambda i: (i, 0)),
            pl.BlockSpec((d, d), lambda i: (0, 0)),
            pl.BlockSpec((1, d), lambda i: (0, 0)),
            pl.BlockSpec((1, d), lambda i: (0, 0)),
            pl.BlockSpec((1, d), lambda i: (0, 0)),
            pl.BlockSpec((d, d), lambda i: (0, 0)),
            pl.BlockSpec((1, d), lambda i: (0, 0)),
        ],
        out_specs=[
            pl.BlockSpec((blk_l, DH), lambda i: (i, 0)),
            pl.BlockSpec((blk_l, DH), lambda i: (i, 0)),
            pl.BlockSpec((blk_p, DH), lambda i: (i, 0)),
            pl.BlockSpec((blk_p, DH), lambda i: (i, 0)),
        ],
        out_shape=[
            jax.ShapeDtypeStruct((n_proj, DH), jnp.float32),
            jax.ShapeDtypeStruct((n_proj, DH), jnp.float32),
            jax.ShapeDtypeStruct((n_sp + PAD, DH), jnp.float32),
            jax.ShapeDtypeStruct((n_sp + PAD, DH), jnp.float32),
        ],
    )(proj_features, prev_scenepoint_features, W_l, b_l[None],
      ln1_w[None], ln1_b[None], W_r, b_r[None])

    agg, den = _edge_stage(hl0, hl1, hr0, hr1, edge_src, edge_dst, att)

    out = pl.pallas_call(
        _final_body,
        grid=(n_sp // blk_r,),
        in_specs=[
            pl.BlockSpec((blk_r, DH), lambda i: (i, 0)),
            pl.BlockSpec((blk_r, DH), lambda i: (i, 0)),
            pl.BlockSpec((blk_r, 16), lambda i: (i, 0)),
            pl.BlockSpec((blk_r, 16), lambda i: (i, 0)),
            pl.BlockSpec((blk_r, d), lambda i: (i, 0)),
            pl.BlockSpec((1, d), lambda i: (0, 0)),
            pl.BlockSpec((1, d), lambda i: (0, 0)),
            pl.BlockSpec((1, d), lambda i: (0, 0)),
            pl.BlockSpec((d, d), lambda i: (0, 0)),
            pl.BlockSpec((1, d), lambda i: (0, 0)),
        ],
        out_specs=pl.BlockSpec((blk_r, d), lambda i: (i, 0)),
        out_shape=jax.ShapeDtypeStruct((n_sp, d), jnp.float32),
    )(agg[0], agg[1], den[0], den[1], prev_scenepoint_features,
      out_bias[None], ln2_w[None], ln2_b[None], W_mlp, b_mlp[None])
    return out
